# baseline (speedup 1.0000x reference)
import functools

import jax
import jax.numpy as jnp
from jax import lax
from jax.experimental import pallas as pl
from jax.experimental.pallas import tpu as pltpu
from jax.experimental.pallas import tpu_sc as plsc

NUM_USERS = 1000000
NUM_MOVIES = 100000
BATCH = 16384
EMB = 64
NUM_CORES = 2
NUM_SUBCORES = 16
NUM_WORKERS = NUM_CORES * NUM_SUBCORES
BPW = BATCH // NUM_WORKERS
NCHUNK = 4
CHUNK = BPW // NCHUNK


def _recommender_sc(up, mp, user_bias, movie_bias, user_idx, movie_idx):
    mesh = plsc.VectorSubcoreMesh(core_axis_name="c", subcore_axis_name="s")

    @functools.partial(
        pl.kernel,
        mesh=mesh,
        compiler_params=pltpu.CompilerParams(needs_layout_passes=False),
        out_type=jax.ShapeDtypeStruct((BATCH,), jnp.float32),
        scratch_types=[
            pltpu.VMEM((NCHUNK, CHUNK), jnp.int32),
            pltpu.VMEM((NCHUNK, CHUNK), jnp.int32),
            pltpu.VMEM((NCHUNK, CHUNK), jnp.int32),
            pltpu.VMEM((NCHUNK, CHUNK), jnp.int32),
            pltpu.VMEM((CHUNK, 2 * EMB), jnp.float32),
            pltpu.VMEM((CHUNK, 2 * EMB), jnp.float32),
            pltpu.VMEM((BPW,), jnp.float32),
            pltpu.VMEM((BPW,), jnp.float32),
            pltpu.VMEM((BPW,), jnp.float32),
            pltpu.SemaphoreType.DMA,
            pltpu.SemaphoreType.DMA,
        ],
    )
    def k(up_hbm, mp_hbm, ubias_hbm, mbias_hbm, uidx_hbm, midx_hbm, out_hbm,
          uidx_v, midx_v, urow_v, mrow_v, urows, mrows,
          ub_v, mb_v, out_v, sem, bsem):
        cid = lax.axis_index("c")
        sid = lax.axis_index("s")
        wid = sid * NUM_CORES + cid
        base = wid * BPW

        for j in range(NCHUNK):
            pltpu.sync_copy(uidx_hbm.at[pl.ds(base + j * CHUNK, CHUNK)], uidx_v.at[j])
            pltpu.sync_copy(midx_hbm.at[pl.ds(base + j * CHUNK, CHUNK)], midx_v.at[j])

        @pl.loop(0, NCHUNK)
        def _(j):
            @pl.loop(0, CHUNK // 16)
            def _(l):
                s = pl.ds(l * 16, 16)
                urow_v[j, s] = lax.shift_right_logical(uidx_v[j, s], 1)
                mrow_v[j, s] = lax.shift_right_logical(midx_v[j, s], 1)

        for j in range(NCHUNK):
            b1 = pltpu.async_copy(ubias_hbm.at[uidx_v.at[j]], ub_v.at[pl.ds(j * CHUNK, CHUNK)], bsem)
            b2 = pltpu.async_copy(mbias_hbm.at[midx_v.at[j]], mb_v.at[pl.ds(j * CHUNK, CHUNK)], bsem)
            b1.wait()
            b2.wait()

        lane = lax.iota(jnp.int32, 16)

        @pl.loop(0, NCHUNK)
        def _(j):
            g1 = pltpu.async_copy(up_hbm.at[urow_v.at[j]], urows, sem)
            g2 = pltpu.async_copy(mp_hbm.at[mrow_v.at[j]], mrows, sem)
            g1.wait()
            g2.wait()

            @pl.loop(0, CHUNK // 16)
            def _(g):
                b0 = g * 16
                s = pl.ds(b0, 16)
                row16 = lane + b0
                uoff = (uidx_v[j, s] & 1) * EMB
                moff = (midx_v[j, s] & 1) * EMB
                acc = (plsc.load_gather(urows, [row16, uoff])
                       * plsc.load_gather(mrows, [row16, moff]))
                for c in range(1, EMB):
                    acc = acc + (plsc.load_gather(urows, [row16, uoff + c])
                                 * plsc.load_gather(mrows, [row16, moff + c]))
                o0 = j * CHUNK + b0
                res = acc + ub_v[pl.ds(o0, 16)] + mb_v[pl.ds(o0, 16)] + 3.5
                out_v[pl.ds(o0, 16)] = res

        pltpu.sync_copy(out_v, out_hbm.at[pl.ds(base, BPW)])

    return k(up, mp, user_bias, movie_bias, user_idx, movie_idx)


def kernel(user_idx, movie_idx, user_embedding, movie_embedding, user_bias, movie_bias):
    return _recommender_sc(
        user_embedding.reshape(NUM_USERS // 2, 2 * EMB),
        movie_embedding.reshape(NUM_MOVIES // 2, 2 * EMB),
        user_bias.reshape(-1),
        movie_bias.reshape(-1),
        user_idx.astype(jnp.int32),
        movie_idx.astype(jnp.int32),
    )


# speedup vs baseline: 1.1321x; 1.1321x over previous
import functools

import jax
import jax.numpy as jnp
from jax import lax
from jax.experimental import pallas as pl
from jax.experimental.pallas import tpu as pltpu
from jax.experimental.pallas import tpu_sc as plsc

NUM_USERS = 1000000
NUM_MOVIES = 100000
BATCH = 16384
EMB = 64
ROW = 128
NUM_CORES = 2
NUM_SUBCORES = 16
NUM_WORKERS = NUM_CORES * NUM_SUBCORES
BPW = BATCH // NUM_WORKERS
NCHUNK = 4
CHUNK = BPW // NCHUNK


def _recommender_sc(up, mp, user_bias, movie_bias, user_idx, movie_idx):
    mesh = plsc.VectorSubcoreMesh(core_axis_name="c", subcore_axis_name="s")

    @functools.partial(
        pl.kernel,
        mesh=mesh,
        compiler_params=pltpu.CompilerParams(
            needs_layout_passes=False, use_tc_tiling_on_sc=True),
        out_type=jax.ShapeDtypeStruct((BATCH,), jnp.float32),
        scratch_types=[
            pltpu.VMEM((NCHUNK, CHUNK), jnp.int32),
            pltpu.VMEM((NCHUNK, CHUNK), jnp.int32),
            pltpu.VMEM((CHUNK, ROW), jnp.float32),
            pltpu.VMEM((CHUNK, ROW), jnp.float32),
            pltpu.VMEM((BPW,), jnp.float32),
            pltpu.VMEM((BPW,), jnp.float32),
            pltpu.VMEM((BPW,), jnp.float32),
            pltpu.VMEM((16, 16), jnp.float32),
            pltpu.SemaphoreType.DMA,
            pltpu.SemaphoreType.DMA,
        ],
    )
    def k(up_hbm, mp_hbm, ubias_hbm, mbias_hbm, uidx_hbm, midx_hbm, out_hbm,
          uidx_v, midx_v, urows, mrows, ub_v, mb_v, out_v, tr_v, sem, bsem):
        cid = lax.axis_index("c")
        sid = lax.axis_index("s")
        wid = sid * NUM_CORES + cid
        base = wid * BPW

        for j in range(NCHUNK):
            pltpu.sync_copy(uidx_hbm.at[pl.ds(base + j * CHUNK, CHUNK)], uidx_v.at[j])
            pltpu.sync_copy(midx_hbm.at[pl.ds(base + j * CHUNK, CHUNK)], midx_v.at[j])

        for j in range(NCHUNK):
            b1 = pltpu.async_copy(ubias_hbm.at[uidx_v.at[j]], ub_v.at[pl.ds(j * CHUNK, CHUNK)], bsem)
            b2 = pltpu.async_copy(mbias_hbm.at[midx_v.at[j]], mb_v.at[pl.ds(j * CHUNK, CHUNK)], bsem)
            b1.wait()
            b2.wait()

        lane = lax.iota(jnp.int32, 16)
        col15 = lane * 0 + 15

        @pl.loop(0, NCHUNK)
        def _(j):
            g1 = pltpu.async_copy(up_hbm.at[uidx_v.at[j]], urows, sem)
            g2 = pltpu.async_copy(mp_hbm.at[midx_v.at[j]], mrows, sem)
            g1.wait()
            g2.wait()

            @pl.loop(0, CHUNK // 16)
            def _(g):
                b0 = g * 16
                for i in range(16):
                    b = b0 + i
                    acc = urows[b, pl.ds(0, 16)] * mrows[b, pl.ds(0, 16)]
                    for c in range(1, 4):
                        acc = acc + (urows[b, pl.ds(c * 16, 16)]
                                     * mrows[b, pl.ds(c * 16, 16)])
                    tr_v[i, :] = jnp.cumsum(acc)
                hsum = plsc.load_gather(tr_v, [lane, col15])
                o0 = j * CHUNK + b0
                res = hsum + ub_v[pl.ds(o0, 16)] + mb_v[pl.ds(o0, 16)] + 3.5
                out_v[pl.ds(o0, 16)] = res

        pltpu.sync_copy(out_v, out_hbm.at[pl.ds(base, BPW)])

    return k(up, mp, user_bias, movie_bias, user_idx, movie_idx)


def kernel(user_idx, movie_idx, user_embedding, movie_embedding, user_bias, movie_bias):
    pad = ((0, 0), (0, ROW - EMB))
    return _recommender_sc(
        jnp.pad(user_embedding, pad),
        jnp.pad(movie_embedding, pad),
        user_bias.reshape(-1),
        movie_bias.reshape(-1),
        user_idx.astype(jnp.int32),
        movie_idx.astype(jnp.int32),
    )
